# baseline (speedup 1.0000x reference)
import functools

import jax
import jax.numpy as jnp
from jax import lax
from jax.experimental import pallas as pl
from jax.experimental.pallas import tpu as pltpu
from jax.experimental.pallas import tpu_sc as plsc

N_DATASETS = 1000
N_CLASSES = 2
DIM = 256
MOMENTUM = 0.99
B = 16384

NSEG = N_DATASETS * N_CLASSES
SEG_PAD = 2048
HROWS = 2 * SEG_PAD
HDIM = 128

NC = 2
NS = 16
L = 16
NW = NC * NS
RPT = B // NW
CHUNK = 64
NCHUNK = RPT // CHUNK
VPR = DIM // L
NBUF = 3

_GDN = lax.GatherDimensionNumbers(
    offset_dims=(), collapsed_slice_dims=(0,), start_index_map=(0,))


def _perm16(x, perm):
    return lax.gather(x, perm[:, None], _GDN, slice_sizes=(1,),
                      mode=lax.GatherScatterMode.PROMISE_IN_BOUNDS)


def _rsqrt16(s):
    i = lax.bitcast_convert_type(s, jnp.int32)
    y = lax.bitcast_convert_type(
        jnp.int32(0x5F3759DF) - lax.shift_right_logical(i, 1), jnp.float32)
    for _ in range(3):
        y = y * (1.5 - 0.5 * s * y * y)
    return jnp.minimum(y, 1e12)


def _transpose16(vs):
    lane = lax.broadcasted_iota(jnp.int32, (L,), 0)
    for s in (1, 2, 4, 8):
        m = (lane & s) == 0
        new = list(vs)
        for i in range(L):
            if i & s == 0:
                j = i | s
                a, b = vs[i], vs[j]
                ap = _perm16(a, lane ^ s)
                bp = _perm16(b, lane ^ s)
                new[i] = jnp.where(m, a, bp)
                new[j] = jnp.where(m, ap, b)
        vs = new
    return vs


def _sc_body(emb_hbm, seg_hbm, psums_hbm, pcnt_hbm,
             buf0, buf1, buf2, idx0, idx1, idx2, hx0, hx1, hx2,
             cnt2d, cnt1d, zbuf, sqb, acc, sem_in, sem_scat):
    cid = lax.axis_index("c")
    sid = lax.axis_index("s")
    wid = cid * NS + sid
    base = wid * RPT

    zero16 = jnp.zeros((L,), jnp.float32)
    ones16 = jnp.ones((L,), jnp.float32)
    iota16 = lax.broadcasted_iota(jnp.int32, (L,), 0)

    for r in range(L):
        for j in range(HDIM // L):
            zbuf[r, pl.ds(j * L, L)] = zero16

    def zero_cnt(j, c):
        for r in range(L):
            cnt2d[r, pl.ds(j * L, L)] = zero16
        return c
    lax.fori_loop(0, SEG_PAD // L, zero_cnt, 0)

    def zero_acc(k, c):
        pltpu.sync_copy(zbuf, acc.at[pl.ds(sid * 256 + k * L, L)])
        return c
    lax.fori_loop(0, 256 // L, zero_acc, 0)

    plsc.subcore_barrier()

    bufs = (buf0, buf1, buf2)
    idxs = (idx0, idx1, idx2)
    hxs = (hx0, hx1, hx2)

    def norm_chunk(buf):
        def row_body(r, c):
            vs = [buf[2 * r + (j // 8), pl.ds((j % 8) * L, L)]
                  for j in range(VPR)]
            sq = [v * v for v in vs]
            while len(sq) > 1:
                sq = [sq[i] + sq[i + 1] for i in range(0, len(sq) - 1, 2)] + \
                     (sq[-1:] if len(sq) % 2 else [])
            sqb[pl.ds(r * L, L)] = sq[0]
            return c
        lax.fori_loop(0, CHUNK, row_body, 0, unroll=2)

        def group_body(g, c):
            ps = [sqb[pl.ds((g * L + rr) * L, L)] for rr in range(L)]
            ts = _transpose16(ps)
            tot = ts[0]
            for rr in range(1, L):
                tot = tot + ts[rr]
            y = _rsqrt16(tot)

            def scale_body(rr, c2):
                yr = _perm16(y, jnp.full((L,), 0, jnp.int32) + rr)
                r = g * L + rr
                for j in range(VPR):
                    h = 2 * r + (j // 8)
                    o = (j % 8) * L
                    buf[h, pl.ds(o, L)] = buf[h, pl.ds(o, L)] * yr
                return c2
            lax.fori_loop(0, L, scale_body, 0, unroll=2)
            return c
        lax.fori_loop(0, CHUNK // L, group_body, 0)

    def build_hx(cidx, hx):
        for g in range(CHUNK // L):
            s16 = cidx[pl.ds(g * L, L)]
            pos = 2 * L * g + 2 * iota16
            plsc.store_scatter(hx, [pos], 2 * s16)
            plsc.store_scatter(hx, [pos + 1], 2 * s16 + 1)

    d_e = [None] * NCHUNK
    d_i = [None] * NCHUNK
    scat = [None] * NCHUNK
    for k in range(min(NBUF - 1, NCHUNK)):
        off = base + k * CHUNK
        d_e[k] = pltpu.async_copy(
            emb_hbm.at[pl.ds(2 * off, 2 * CHUNK)], bufs[k % NBUF], sem_in)
        d_i[k] = pltpu.async_copy(
            seg_hbm.at[pl.ds(off, CHUNK)], idxs[k % NBUF], sem_in)
    for k in range(NCHUNK):
        if k + NBUF - 1 < NCHUNK:
            kk = k + NBUF - 1
            if kk - NBUF >= 0:
                scat[kk - NBUF].wait()
            off = base + kk * CHUNK
            d_e[kk] = pltpu.async_copy(
                emb_hbm.at[pl.ds(2 * off, 2 * CHUNK)],
                bufs[kk % NBUF], sem_in)
            d_i[kk] = pltpu.async_copy(
                seg_hbm.at[pl.ds(off, CHUNK)], idxs[kk % NBUF], sem_in)
        d_e[k].wait()
        d_i[k].wait()
        cur = bufs[k % NBUF]
        cidx = idxs[k % NBUF]
        chx = hxs[k % NBUF]

        norm_chunk(cur)
        build_hx(cidx, chx)

        for g in range(CHUNK // L):
            seg16 = cidx[pl.ds(g * L, L)]
            plsc.addupdate_scatter(cnt2d, [iota16, seg16], ones16)

        scat[k] = pltpu.async_copy(cur, acc.at[chx], sem_scat, add=True)
    for k in range(max(0, NCHUNK - NBUF), NCHUNK):
        if scat[k] is not None:
            scat[k].wait()

    plsc.subcore_barrier()

    def cnt_red(j, c):
        a = cnt2d[0, pl.ds(j * L, L)]
        for r in range(1, L):
            a = a + cnt2d[r, pl.ds(j * L, L)]
        pos = 2 * (j * L + iota16)
        plsc.store_scatter(cnt1d, [pos], a)
        plsc.store_scatter(cnt1d, [pos + 1], a)
        return c
    lax.fori_loop(0, SEG_PAD // L, cnt_red, 0)

    pltpu.sync_copy(cnt1d, pcnt_hbm.at[wid])
    pltpu.sync_copy(acc.at[pl.ds(sid * 256, 256)],
                    psums_hbm.at[cid, pl.ds(sid * 256, 256)])


_sc_segsum = functools.partial(
    pl.kernel,
    out_type=(
        jax.ShapeDtypeStruct((NC, HROWS, HDIM), jnp.float32),
        jax.ShapeDtypeStruct((NW, HROWS), jnp.float32),
    ),
    mesh=plsc.VectorSubcoreMesh(core_axis_name="c", subcore_axis_name="s"),
    compiler_params=pltpu.CompilerParams(use_tc_tiling_on_sc=False,
                                         needs_layout_passes=False),
    scratch_types=[
        pltpu.VMEM((2 * CHUNK, HDIM), jnp.float32),
        pltpu.VMEM((2 * CHUNK, HDIM), jnp.float32),
        pltpu.VMEM((2 * CHUNK, HDIM), jnp.float32),
        pltpu.VMEM((CHUNK,), jnp.int32),
        pltpu.VMEM((CHUNK,), jnp.int32),
        pltpu.VMEM((CHUNK,), jnp.int32),
        pltpu.VMEM((2 * CHUNK,), jnp.int32),
        pltpu.VMEM((2 * CHUNK,), jnp.int32),
        pltpu.VMEM((2 * CHUNK,), jnp.int32),
        pltpu.VMEM((L, SEG_PAD), jnp.float32),
        pltpu.VMEM((HROWS,), jnp.float32),
        pltpu.VMEM((L, HDIM), jnp.float32),
        pltpu.VMEM((CHUNK * L,), jnp.float32),
        pltpu.VMEM_SHARED((HROWS, HDIM), jnp.float32),
        pltpu.SemaphoreType.DMA,
        pltpu.SemaphoreType.DMA,
    ],
)(_sc_body)


def _epi_body(ps_ref, pc_ref, proto_ref, initf_ref, out_g_ref, out_p_ref):
    sums = ps_ref[0] + ps_ref[1]
    counts_h = jnp.sum(pc_ref[...], axis=0).reshape(HROWS, 1)
    protos = jnp.concatenate(
        [proto_ref[...],
         jnp.zeros((HROWS - 2 * NSEG, HDIM), jnp.float32)], axis=0)
    initf_h = initf_ref[...]

    has_h = (counts_h >= 1.0).astype(jnp.float32)

    bp = sums / jnp.maximum(counts_h, 1.0)
    ema = MOMENTUM * protos + (1.0 - MOMENTUM) * bp
    upd = jnp.where(initf_h > 0.0, ema, bp)
    newp = jnp.where(has_h > 0.0, upd, protos)
    out_p_ref[...] = newp

    new_initf_h = jnp.maximum(initf_h, has_h)

    hrow = lax.broadcasted_iota(jnp.int32, (HROWS, 1), 0)
    is_h0 = hrow % 2 == 0
    half0 = is_h0.astype(jnp.float32)
    half1 = 1.0 - half0

    nh = jnp.sum(newp * newp, axis=1, keepdims=True)
    partner = jnp.where(is_h0, jnp.roll(nh, -1, axis=0),
                        jnp.roll(nh, 1, axis=0))
    npair = nh + partner
    inv = 1.0 / jnp.maximum(jnp.sqrt(npair), 1e-12)
    pn = newp * inv

    seg_of = hrow // 2
    validf = (seg_of < NSEG).astype(jnp.float32)
    cls0_h = (seg_of % 2 == 0).astype(jnp.float32) * validf
    cls1_h = (seg_of % 2 == 1).astype(jnp.float32) * validf

    w0 = cls0_h * new_initf_h
    w1 = cls1_h * new_initf_h
    masks = jnp.concatenate(
        [w0 * half0, w0 * half1, w1 * half0, w1 * half1], axis=1)
    quad = jax.lax.dot_general(
        masks, pn, (((0,), (0,)), ((), ())),
        precision=lax.Precision.HIGHEST,
        preferred_element_type=jnp.float32)
    den0 = jnp.maximum(jnp.sum(w0 * half0), 1.0)
    den1 = jnp.maximum(jnp.sum(w1 * half0), 1.0)
    g = jnp.concatenate(
        [jnp.concatenate([quad[0:1], quad[1:2]], axis=1) / den0,
         jnp.concatenate([quad[2:3], quad[3:4]], axis=1) / den1],
        axis=0)
    g_norm = jnp.sqrt(jnp.sum(g * g, axis=1, keepdims=True))
    out_g_ref[...] = g / jnp.maximum(g_norm, 1e-12)


def _epilogue(psums, pcnts, protos_h, initf):
    return pl.pallas_call(
        _epi_body,
        out_shape=[
            jax.ShapeDtypeStruct((N_CLASSES, DIM), jnp.float32),
            jax.ShapeDtypeStruct((HROWS, HDIM), jnp.float32),
        ],
    )(psums, pcnts, protos_h, initf)


@jax.jit
def _run(emb_h, seg, protos_h, initf):
    psums, pcnts = _sc_segsum(emb_h, seg)
    return _epilogue(psums, pcnts, protos_h, initf)


def kernel(embeddings, labels, dataset_ids, prototypes, initialized):
    seg = dataset_ids.astype(jnp.int32) * N_CLASSES + labels.astype(jnp.int32)
    emb_h = embeddings.astype(jnp.float32).reshape(2 * B, HDIM)
    protos_h = prototypes.reshape(2 * NSEG, HDIM)
    initf_h = jnp.pad(
        jnp.repeat(initialized.reshape(NSEG).astype(jnp.float32), 2),
        (0, HROWS - 2 * NSEG)).reshape(HROWS, 1)
    g, newp_h = _run(emb_h, seg, protos_h, initf_h)
    return (g, newp_h[:2 * NSEG].reshape(N_DATASETS, N_CLASSES, DIM))


# speedup vs baseline: 1.0943x; 1.0943x over previous
import functools

import jax
import jax.numpy as jnp
from jax import lax
from jax.experimental import pallas as pl
from jax.experimental.pallas import tpu as pltpu
from jax.experimental.pallas import tpu_sc as plsc

N_DATASETS = 1000
N_CLASSES = 2
DIM = 256
MOMENTUM = 0.99
B = 16384

NSEG = N_DATASETS * N_CLASSES
SEG_PAD = 2048

NC = 2
NS = 16
L = 16
NW = NC * NS
RPT = B // NW
CHUNK = 64
NCHUNK = RPT // CHUNK
VPR = DIM // L
NBUF = 3


_GDN = lax.GatherDimensionNumbers(
    offset_dims=(), collapsed_slice_dims=(0,), start_index_map=(0,))


def _perm16(x, perm):
    return lax.gather(x, perm[:, None], _GDN, slice_sizes=(1,),
                      mode=lax.GatherScatterMode.PROMISE_IN_BOUNDS)


def _rsqrt16(s):
    i = lax.bitcast_convert_type(s, jnp.int32)
    y = lax.bitcast_convert_type(
        jnp.int32(0x5F3759DF) - lax.shift_right_logical(i, 1), jnp.float32)
    for _ in range(3):
        y = y * (1.5 - 0.5 * s * y * y)
    return jnp.minimum(y, 1e12)


def _transpose16(vs):
    lane = lax.broadcasted_iota(jnp.int32, (L,), 0)
    for s in (1, 2, 4, 8):
        m = (lane & s) == 0
        new = list(vs)
        for i in range(L):
            if i & s == 0:
                j = i | s
                a, b = vs[i], vs[j]
                ap = _perm16(a, lane ^ s)
                bp = _perm16(b, lane ^ s)
                new[i] = jnp.where(m, a, bp)
                new[j] = jnp.where(m, ap, b)
        vs = new
    return vs


def _sc_body(emb_hbm, seg_hbm, psums_hbm, pcnt_hbm,
             buf0, buf1, buf2, idx0, idx1, idx2, cnt2d, cnt1d, zbuf, sqb, acc,
             sem_in, sem_scat):
    cid = lax.axis_index("c")
    sid = lax.axis_index("s")
    wid = cid * NS + sid
    base = wid * RPT

    zero16 = jnp.zeros((L,), jnp.float32)
    ones16 = jnp.ones((L,), jnp.float32)
    iota16 = lax.broadcasted_iota(jnp.int32, (L,), 0)

    for r in range(L):
        for j in range(VPR):
            zbuf[r, pl.ds(j * L, L)] = zero16

    def zero_cnt(j, c):
        for r in range(L):
            cnt2d[r, pl.ds(j * L, L)] = zero16
        return c
    lax.fori_loop(0, SEG_PAD // L, zero_cnt, 0)

    def zero_acc(k, c):
        pltpu.sync_copy(zbuf, acc.at[pl.ds(sid * 128 + k * L, L)])
        return c
    lax.fori_loop(0, 128 // L, zero_acc, 0)

    plsc.subcore_barrier()

    bufs = (buf0, buf1, buf2)
    idxs = (idx0, idx1, idx2)

    def norm_chunk(buf):
        def row_body(r, c):
            vs = [buf[r, pl.ds(j * L, L)] for j in range(VPR)]
            sq = [v * v for v in vs]
            while len(sq) > 1:
                sq = [sq[i] + sq[i + 1] for i in range(0, len(sq) - 1, 2)] + \
                     (sq[-1:] if len(sq) % 2 else [])
            t = sq[0]
            for sh in (8, 4, 2, 1):
                t = t + _perm16(t, iota16 ^ sh)
            y = _rsqrt16(t)
            for j in range(VPR):
                buf[r, pl.ds(j * L, L)] = vs[j] * y
            return c
        lax.fori_loop(0, CHUNK, row_body, 0, unroll=2)

    d_e = [None] * NCHUNK
    d_i = [None] * NCHUNK
    scat = [None] * NCHUNK
    for k in range(min(NBUF - 1, NCHUNK)):
        off = base + k * CHUNK
        d_e[k] = pltpu.async_copy(
            emb_hbm.at[pl.ds(off, CHUNK)], bufs[k % NBUF], sem_in)
        d_i[k] = pltpu.async_copy(
            seg_hbm.at[pl.ds(off, CHUNK)], idxs[k % NBUF], sem_in)
    for k in range(NCHUNK):
        if k + NBUF - 1 < NCHUNK:
            kk = k + NBUF - 1
            if kk - NBUF >= 0:
                scat[kk - NBUF].wait()
            off = base + kk * CHUNK
            d_e[kk] = pltpu.async_copy(
                emb_hbm.at[pl.ds(off, CHUNK)], bufs[kk % NBUF], sem_in)
            d_i[kk] = pltpu.async_copy(
                seg_hbm.at[pl.ds(off, CHUNK)], idxs[kk % NBUF], sem_in)
        d_e[k].wait()
        d_i[k].wait()
        cur = bufs[k % NBUF]
        cidx = idxs[k % NBUF]

        norm_chunk(cur)

        for g in range(CHUNK // L):
            seg16 = cidx[pl.ds(g * L, L)]
            plsc.addupdate_scatter(cnt2d, [iota16, seg16], ones16)

        scat[k] = pltpu.async_copy(cur, acc.at[cidx], sem_scat, add=True)
    for k in range(max(0, NCHUNK - NBUF), NCHUNK):
        if scat[k] is not None:
            scat[k].wait()

    plsc.subcore_barrier()

    def cnt_red(j, c):
        a = cnt2d[0, pl.ds(j * L, L)]
        for r in range(1, L):
            a = a + cnt2d[r, pl.ds(j * L, L)]
        cnt1d[pl.ds(j * L, L)] = a
        return c
    lax.fori_loop(0, SEG_PAD // L, cnt_red, 0)

    pltpu.sync_copy(cnt1d, pcnt_hbm.at[wid])
    pltpu.sync_copy(acc.at[pl.ds(sid * 128, 128)],
                    psums_hbm.at[cid, pl.ds(sid * 128, 128)])


_sc_segsum = functools.partial(
    pl.kernel,
    out_type=(
        jax.ShapeDtypeStruct((NC, SEG_PAD, DIM), jnp.float32),
        jax.ShapeDtypeStruct((NW, SEG_PAD), jnp.float32),
    ),
    mesh=plsc.VectorSubcoreMesh(core_axis_name="c", subcore_axis_name="s"),
    compiler_params=pltpu.CompilerParams(use_tc_tiling_on_sc=False,
                                         needs_layout_passes=False),
    scratch_types=[
        pltpu.VMEM((CHUNK, DIM), jnp.float32),
        pltpu.VMEM((CHUNK, DIM), jnp.float32),
        pltpu.VMEM((CHUNK, DIM), jnp.float32),
        pltpu.VMEM((CHUNK,), jnp.int32),
        pltpu.VMEM((CHUNK,), jnp.int32),
        pltpu.VMEM((CHUNK,), jnp.int32),
        pltpu.VMEM((L, SEG_PAD), jnp.float32),
        pltpu.VMEM((SEG_PAD,), jnp.float32),
        pltpu.VMEM((L, DIM), jnp.float32),
        pltpu.VMEM((CHUNK * L,), jnp.float32),
        pltpu.VMEM_SHARED((SEG_PAD, DIM), jnp.float32),
        pltpu.SemaphoreType.DMA,
        pltpu.SemaphoreType.DMA,
    ],
)(_sc_body)


def _epi_body(ps_ref, pc_ref, proto_ref, initf_ref, out_g_ref, out_p_ref):
    sums = ps_ref[0] + ps_ref[1]
    counts = jnp.sum(pc_ref[...], axis=0).reshape(SEG_PAD, 1)
    protos = proto_ref[...]
    initf = initf_ref[...]

    has = (counts >= 1.0).astype(jnp.float32)
    bp = sums / jnp.maximum(counts, 1.0)
    ema = MOMENTUM * protos + (1.0 - MOMENTUM) * bp
    upd = jnp.where(initf > 0.0, ema, bp)
    newp = jnp.where(has > 0.0, upd, protos)
    out_p_ref[...] = newp[:NSEG, :]

    new_initf = jnp.maximum(initf, has)

    pn_norm = jnp.sqrt(jnp.sum(newp * newp, axis=1, keepdims=True))
    pn = newp / jnp.maximum(pn_norm, 1e-12)

    rows = lax.broadcasted_iota(jnp.int32, (SEG_PAD, 1), 0)
    valid = (rows < NSEG).astype(jnp.float32)
    even = (rows % 2 == 0).astype(jnp.float32) * valid
    odd = (rows % 2 == 1).astype(jnp.float32) * valid

    w0 = new_initf * even
    w1 = new_initf * odd
    num0 = jnp.sum(pn * w0, axis=0, keepdims=True)
    num1 = jnp.sum(pn * w1, axis=0, keepdims=True)
    den0 = jnp.maximum(jnp.sum(w0), 1.0)
    den1 = jnp.maximum(jnp.sum(w1), 1.0)
    g = jnp.concatenate([num0 / den0, num1 / den1], axis=0)
    g_norm = jnp.sqrt(jnp.sum(g * g, axis=1, keepdims=True))
    out_g_ref[...] = g / jnp.maximum(g_norm, 1e-12)


def _epilogue(psums, pcnts, protos_pad, initf):
    return pl.pallas_call(
        _epi_body,
        out_shape=[
            jax.ShapeDtypeStruct((N_CLASSES, DIM), jnp.float32),
            jax.ShapeDtypeStruct((NSEG, DIM), jnp.float32),
        ],
    )(psums, pcnts, protos_pad, initf)


@jax.jit
def _run(embeddings, seg, protos_pad, initf):
    psums, pcnts = _sc_segsum(embeddings, seg)
    return _epilogue(psums, pcnts, protos_pad, initf)


def kernel(embeddings, labels, dataset_ids, prototypes, initialized):
    seg = dataset_ids.astype(jnp.int32) * N_CLASSES + labels.astype(jnp.int32)
    protos2 = prototypes.reshape(NSEG, DIM)
    protos_pad = jnp.pad(protos2, ((0, SEG_PAD - NSEG), (0, 0)))
    initf = jnp.pad(initialized.reshape(NSEG).astype(jnp.float32),
                    (0, SEG_PAD - NSEG)).reshape(SEG_PAD, 1)
    g, newp = _run(embeddings.astype(jnp.float32), seg, protos_pad, initf)
    return (g, newp.reshape(N_DATASETS, N_CLASSES, DIM))


# speedup vs baseline: 1.2449x; 1.1376x over previous
import functools

import jax
import jax.numpy as jnp
from jax import lax
from jax.experimental import pallas as pl
from jax.experimental.pallas import tpu as pltpu
from jax.experimental.pallas import tpu_sc as plsc

N_DATASETS = 1000
N_CLASSES = 2
DIM = 256
MOMENTUM = 0.99
B = 16384

NSEG = N_DATASETS * N_CLASSES
SEG_PAD = 2048

NC = 2
NS = 16
L = 16
NW = NC * NS
SC_ROWS = B // 2
TC_ROWS = B - SC_ROWS
RPT = SC_ROWS // NW
CHUNK = 64
NCHUNK = RPT // CHUNK
VPR = DIM // L
NBUF = 3
TROWS = 2048
TNB = TC_ROWS // TROWS


_GDN = lax.GatherDimensionNumbers(
    offset_dims=(), collapsed_slice_dims=(0,), start_index_map=(0,))


def _perm16(x, perm):
    return lax.gather(x, perm[:, None], _GDN, slice_sizes=(1,),
                      mode=lax.GatherScatterMode.PROMISE_IN_BOUNDS)


def _rsqrt16(s):
    i = lax.bitcast_convert_type(s, jnp.int32)
    y = lax.bitcast_convert_type(
        jnp.int32(0x5F3759DF) - lax.shift_right_logical(i, 1), jnp.float32)
    for _ in range(3):
        y = y * (1.5 - 0.5 * s * y * y)
    return jnp.minimum(y, 1e12)


def _transpose16(vs):
    lane = lax.broadcasted_iota(jnp.int32, (L,), 0)
    for s in (1, 2, 4, 8):
        m = (lane & s) == 0
        new = list(vs)
        for i in range(L):
            if i & s == 0:
                j = i | s
                a, b = vs[i], vs[j]
                ap = _perm16(a, lane ^ s)
                bp = _perm16(b, lane ^ s)
                new[i] = jnp.where(m, a, bp)
                new[j] = jnp.where(m, ap, b)
        vs = new
    return vs


def _sc_body(emb_hbm, seg_hbm, psums_hbm, pcnt_hbm,
             buf0, buf1, buf2, idx0, idx1, idx2, cnt2d, cnt1d, zbuf, sqb, acc,
             sem_in, sem_scat):
    cid = lax.axis_index("c")
    sid = lax.axis_index("s")
    wid = cid * NS + sid
    base = wid * RPT

    zero16 = jnp.zeros((L,), jnp.float32)
    ones16 = jnp.ones((L,), jnp.float32)
    iota16 = lax.broadcasted_iota(jnp.int32, (L,), 0)

    for r in range(L):
        for j in range(VPR):
            zbuf[r, pl.ds(j * L, L)] = zero16

    def zero_cnt(j, c):
        for r in range(L):
            cnt2d[r, pl.ds(j * L, L)] = zero16
        return c
    lax.fori_loop(0, SEG_PAD // L, zero_cnt, 0)

    def zero_acc(k, c):
        pltpu.sync_copy(zbuf, acc.at[pl.ds(sid * 128 + k * L, L)])
        return c
    lax.fori_loop(0, 128 // L, zero_acc, 0)

    plsc.subcore_barrier()

    bufs = (buf0, buf1, buf2)
    idxs = (idx0, idx1, idx2)

    def norm_chunk(buf):
        def row_body(r, c):
            vs = [buf[r, pl.ds(j * L, L)] for j in range(VPR)]
            sq = [v * v for v in vs]
            while len(sq) > 1:
                sq = [sq[i] + sq[i + 1] for i in range(0, len(sq) - 1, 2)] + \
                     (sq[-1:] if len(sq) % 2 else [])
            t = sq[0]
            for sh in (8, 4, 2, 1):
                t = t + _perm16(t, iota16 ^ sh)
            y = _rsqrt16(t)
            for j in range(VPR):
                buf[r, pl.ds(j * L, L)] = vs[j] * y
            return c
        lax.fori_loop(0, CHUNK, row_body, 0, unroll=2)

    d_e = [None] * NCHUNK
    d_i = [None] * NCHUNK
    scat = [None] * NCHUNK
    for k in range(min(NBUF - 1, NCHUNK)):
        off = base + k * CHUNK
        d_e[k] = pltpu.async_copy(
            emb_hbm.at[pl.ds(off, CHUNK)], bufs[k % NBUF], sem_in)
        d_i[k] = pltpu.async_copy(
            seg_hbm.at[pl.ds(off, CHUNK)], idxs[k % NBUF], sem_in)
    for k in range(NCHUNK):
        if k + NBUF - 1 < NCHUNK:
            kk = k + NBUF - 1
            if kk - NBUF >= 0:
                scat[kk - NBUF].wait()
            off = base + kk * CHUNK
            d_e[kk] = pltpu.async_copy(
                emb_hbm.at[pl.ds(off, CHUNK)], bufs[kk % NBUF], sem_in)
            d_i[kk] = pltpu.async_copy(
                seg_hbm.at[pl.ds(off, CHUNK)], idxs[kk % NBUF], sem_in)
        d_e[k].wait()
        d_i[k].wait()
        cur = bufs[k % NBUF]
        cidx = idxs[k % NBUF]

        norm_chunk(cur)

        for g in range(CHUNK // L):
            seg16 = cidx[pl.ds(g * L, L)]
            plsc.addupdate_scatter(cnt2d, [iota16, seg16], ones16)

        scat[k] = pltpu.async_copy(cur, acc.at[cidx], sem_scat, add=True)
    for k in range(max(0, NCHUNK - NBUF), NCHUNK):
        if scat[k] is not None:
            scat[k].wait()

    plsc.subcore_barrier()

    def cnt_red(j, c):
        a = cnt2d[0, pl.ds(j * L, L)]
        for r in range(1, L):
            a = a + cnt2d[r, pl.ds(j * L, L)]
        cnt1d[pl.ds(j * L, L)] = a
        return c
    lax.fori_loop(0, SEG_PAD // L, cnt_red, 0)

    pltpu.sync_copy(cnt1d, pcnt_hbm.at[wid])
    pltpu.sync_copy(acc.at[pl.ds(sid * 128, 128)],
                    psums_hbm.at[cid, pl.ds(sid * 128, 128)])


_sc_segsum = functools.partial(
    pl.kernel,
    out_type=(
        jax.ShapeDtypeStruct((NC, SEG_PAD, DIM), jnp.float32),
        jax.ShapeDtypeStruct((NW, SEG_PAD), jnp.float32),
    ),
    mesh=plsc.VectorSubcoreMesh(core_axis_name="c", subcore_axis_name="s"),
    compiler_params=pltpu.CompilerParams(use_tc_tiling_on_sc=False,
                                         needs_layout_passes=False),
    scratch_types=[
        pltpu.VMEM((CHUNK, DIM), jnp.float32),
        pltpu.VMEM((CHUNK, DIM), jnp.float32),
        pltpu.VMEM((CHUNK, DIM), jnp.float32),
        pltpu.VMEM((CHUNK,), jnp.int32),
        pltpu.VMEM((CHUNK,), jnp.int32),
        pltpu.VMEM((CHUNK,), jnp.int32),
        pltpu.VMEM((L, SEG_PAD), jnp.float32),
        pltpu.VMEM((SEG_PAD,), jnp.float32),
        pltpu.VMEM((L, DIM), jnp.float32),
        pltpu.VMEM((CHUNK * L,), jnp.float32),
        pltpu.VMEM_SHARED((SEG_PAD, DIM), jnp.float32),
        pltpu.SemaphoreType.DMA,
        pltpu.SemaphoreType.DMA,
    ],
)(_sc_body)


def _tc_seg_body(emb_ref, seg_ref, out_s_ref, out_c_ref, sums_ref, counts_ref):
    i = pl.program_id(0)

    @pl.when(i == 0)
    def _init():
        sums_ref[...] = jnp.zeros_like(sums_ref)
        counts_ref[...] = jnp.zeros_like(counts_ref)

    emb = emb_ref[...]
    seg = seg_ref[0]

    norm = jnp.sqrt(jnp.sum(emb * emb, axis=1, keepdims=True))
    emb_n = emb / jnp.maximum(norm, 1e-12)

    seg_ids = jax.lax.broadcasted_iota(jnp.int32, (SEG_PAD, TROWS), 0)
    onehot = (seg_ids == seg).astype(jnp.float32)
    sums_ref[...] += jnp.dot(onehot, emb_n,
                             preferred_element_type=jnp.float32)
    counts_ref[...] += jnp.sum(onehot, axis=1, keepdims=True)

    @pl.when(i == TNB - 1)
    def _done():
        out_s_ref[...] = sums_ref[...]
        out_c_ref[...] = counts_ref[...]


def _tc_partial(embeddings, seg3):
    skip = SC_ROWS // TROWS
    return pl.pallas_call(
        _tc_seg_body,
        grid=(TNB,),
        in_specs=[
            pl.BlockSpec((TROWS, DIM), lambda i: (i + skip, 0)),
            pl.BlockSpec((1, 1, TROWS), lambda i: (i + skip, 0, 0)),
        ],
        out_specs=[
            pl.BlockSpec((SEG_PAD, DIM), lambda i: (0, 0)),
            pl.BlockSpec((SEG_PAD, 1), lambda i: (0, 0)),
        ],
        out_shape=[
            jax.ShapeDtypeStruct((SEG_PAD, DIM), jnp.float32),
            jax.ShapeDtypeStruct((SEG_PAD, 1), jnp.float32),
        ],
        scratch_shapes=[
            pltpu.VMEM((SEG_PAD, DIM), jnp.float32),
            pltpu.VMEM((SEG_PAD, 1), jnp.float32),
        ],
        compiler_params=pltpu.CompilerParams(
            dimension_semantics=("arbitrary",),
        ),
    )(embeddings, seg3)


def _epi_body(ps_ref, pc_ref, ts_ref, tc_ref, proto_ref, initf_ref,
              out_g_ref, out_p_ref):
    sums = ps_ref[0] + ps_ref[1] + ts_ref[...]
    counts = jnp.sum(pc_ref[...], axis=0).reshape(SEG_PAD, 1) + tc_ref[...]
    protos = proto_ref[...]
    initf = initf_ref[...]

    has = (counts >= 1.0).astype(jnp.float32)
    bp = sums / jnp.maximum(counts, 1.0)
    ema = MOMENTUM * protos + (1.0 - MOMENTUM) * bp
    upd = jnp.where(initf > 0.0, ema, bp)
    newp = jnp.where(has > 0.0, upd, protos)
    out_p_ref[...] = newp[:NSEG, :]

    new_initf = jnp.maximum(initf, has)

    pn_norm = jnp.sqrt(jnp.sum(newp * newp, axis=1, keepdims=True))
    pn = newp / jnp.maximum(pn_norm, 1e-12)

    rows = lax.broadcasted_iota(jnp.int32, (SEG_PAD, 1), 0)
    valid = (rows < NSEG).astype(jnp.float32)
    even = (rows % 2 == 0).astype(jnp.float32) * valid
    odd = (rows % 2 == 1).astype(jnp.float32) * valid

    w0 = new_initf * even
    w1 = new_initf * odd
    num0 = jnp.sum(pn * w0, axis=0, keepdims=True)
    num1 = jnp.sum(pn * w1, axis=0, keepdims=True)
    den0 = jnp.maximum(jnp.sum(w0), 1.0)
    den1 = jnp.maximum(jnp.sum(w1), 1.0)
    g = jnp.concatenate([num0 / den0, num1 / den1], axis=0)
    g_norm = jnp.sqrt(jnp.sum(g * g, axis=1, keepdims=True))
    out_g_ref[...] = g / jnp.maximum(g_norm, 1e-12)


def _epilogue(psums, pcnts, tsums, tcounts, protos_pad, initf):
    return pl.pallas_call(
        _epi_body,
        out_shape=[
            jax.ShapeDtypeStruct((N_CLASSES, DIM), jnp.float32),
            jax.ShapeDtypeStruct((NSEG, DIM), jnp.float32),
        ],
    )(psums, pcnts, tsums, tcounts, protos_pad, initf)


@jax.jit
def _run(embeddings, seg, protos_pad, initf):
    seg3 = seg.reshape(B // TROWS, 1, TROWS)
    psums, pcnts = _sc_segsum(embeddings[:SC_ROWS], seg[:SC_ROWS])
    tsums, tcounts = _tc_partial(embeddings, seg3)
    return _epilogue(psums, pcnts, tsums, tcounts, protos_pad, initf)


def kernel(embeddings, labels, dataset_ids, prototypes, initialized):
    seg = dataset_ids.astype(jnp.int32) * N_CLASSES + labels.astype(jnp.int32)
    protos2 = prototypes.reshape(NSEG, DIM)
    protos_pad = jnp.pad(protos2, ((0, SEG_PAD - NSEG), (0, 0)))
    initf = jnp.pad(initialized.reshape(NSEG).astype(jnp.float32),
                    (0, SEG_PAD - NSEG)).reshape(SEG_PAD, 1)
    g, newp = _run(embeddings.astype(jnp.float32), seg, protos_pad, initf)
    return (g, newp.reshape(N_DATASETS, N_CLASSES, DIM))
